# baseline (speedup 1.0000x reference)
import functools

import jax
import jax.numpy as jnp
from jax import lax
from jax.experimental import pallas as pl
from jax.experimental.pallas import tpu as pltpu
from jax.experimental.pallas import tpu_sc as plsc

B = 16384
D = 64
NC = 2
NS = 16
NW = NC * NS
BPW = B // NW
CH = 16
NCHUNK = BPW // CH
TROWS = 8


def _scalar(vec, lane_iota, e):
    return jnp.max(jnp.where(lane_iota == e, vec, jnp.int32(-1)))


def _body(x_h, tab, out, x_v, tiles, rows_c, sem):
    wid = lax.axis_index("s") * NC + lax.axis_index("c")
    base = wid * BPW
    pltpu.sync_copy(x_h.at[pl.ds(base, BPW)], x_v)
    lane_iota = lax.iota(jnp.int32, 16)

    def chunk(c, _):
        vec = x_v[pl.ds(c * CH, CH)]
        rs = [_scalar(vec, lane_iota, e) for e in range(CH)]
        copies = []
        for e in range(CH):
            copies.append(pltpu.async_copy(
                tab.at[pl.ds(rs[e] >> 3, 1)], tiles.at[pl.ds(e, 1)], sem))
        for cp in copies:
            cp.wait()
        for e in range(CH):
            su = rs[e] & 7
            for q in range(D // 16):
                rows_c[e, pl.ds(16 * q, 16)] = tiles[e, su, pl.ds(16 * q, 16)]
        pltpu.sync_copy(rows_c, out.at[pl.ds(base + c * CH, CH)])
        return ()

    lax.fori_loop(0, NCHUNK, chunk, (), unroll=False)


def _make_lookup():
    mesh = plsc.VectorSubcoreMesh(core_axis_name="c", subcore_axis_name="s")
    return functools.partial(
        pl.kernel,
        mesh=mesh,
        out_type=jax.ShapeDtypeStruct((B, D), jnp.float32),
        scratch_types=[
            pltpu.VMEM((BPW,), jnp.int32),
            pltpu.VMEM((CH, TROWS, D), jnp.float32),
            pltpu.VMEM((CH, D), jnp.float32),
            pltpu.SemaphoreType.DMA,
        ],
        compiler_params=pltpu.CompilerParams(needs_layout_passes=False),
    )(_body)


@jax.jit
def _lookup2(xu, xv, uid_t, iid_t):
    ou = _make_lookup()(xu, uid_t)
    oi = _make_lookup()(xv, iid_t)
    return ou, oi


def kernel(x, uid_table, iid_table):
    xi = x.astype(jnp.int32)
    ut = uid_table.reshape(125000, TROWS, D)
    it = iid_table[:1000000].reshape(125000, TROWS, D)
    ou, oi = _lookup2(xi[:, 0], xi[:, 1], ut, it)
    return jnp.stack([ou, oi], axis=1)


# speedup vs baseline: 1.1069x; 1.1069x over previous
import functools

import jax
import jax.numpy as jnp
from jax import lax
from jax.experimental import pallas as pl
from jax.experimental.pallas import tpu as pltpu
from jax.experimental.pallas import tpu_sc as plsc

B = 16384
D = 64
NC = 2
NS = 16
NW = NC * NS
BPW = B // NW
CH = 16
NCHUNK = BPW // CH
TROWS = 8


def _scalar(vec, lane_iota, e):
    return jnp.max(jnp.where(lane_iota == e, vec, jnp.int32(-1)))


def _body(xu_h, xv_h, uid_tab, iid_tab, out,
          xu_v, xv_v, tu0, ti0, tu1, ti1, rows_c, sem0, sem1):
    wid = lax.axis_index("s") * NC + lax.axis_index("c")
    base = wid * BPW
    pltpu.sync_copy(xu_h.at[pl.ds(base, BPW)], xu_v)
    pltpu.sync_copy(xv_h.at[pl.ds(base, BPW)], xv_v)
    lane_iota = lax.iota(jnp.int32, 16)

    def issue(c, tu, ti, sem):
        vec_u = xu_v[pl.ds(c * CH, CH)]
        vec_i = xv_v[pl.ds(c * CH, CH)]
        rus = [_scalar(vec_u, lane_iota, e) for e in range(CH)]
        ris = [_scalar(vec_i, lane_iota, e) for e in range(CH)]
        copies = []
        for e in range(CH):
            copies.append(pltpu.async_copy(
                uid_tab.at[pl.ds(rus[e] >> 3, 1)], tu.at[pl.ds(e, 1)], sem))
            copies.append(pltpu.async_copy(
                iid_tab.at[pl.ds(ris[e] >> 3, 1)], ti.at[pl.ds(e, 1)], sem))
        return rus, ris, copies

    def extract(c, rus, ris, tu, ti):
        for e in range(CH):
            su = rus[e] & 7
            si = ris[e] & 7
            for q in range(D // 16):
                rows_c[e, pl.ds(16 * q, 16)] = tu[e, su, pl.ds(16 * q, 16)]
                rows_c[e, pl.ds(D + 16 * q, 16)] = ti[e, si, pl.ds(16 * q, 16)]
        pltpu.sync_copy(rows_c, out.at[pl.ds(base + c * CH, CH)])

    def pair(p, _):
        c0 = 2 * p
        c1 = 2 * p + 1
        rus0, ris0, cp0 = issue(c0, tu0, ti0, sem0)
        rus1, ris1, cp1 = issue(c1, tu1, ti1, sem1)
        for cp in cp0:
            cp.wait()
        extract(c0, rus0, ris0, tu0, ti0)
        for cp in cp1:
            cp.wait()
        extract(c1, rus1, ris1, tu1, ti1)
        return ()

    lax.fori_loop(0, NCHUNK // 2, pair, (), unroll=False)


@jax.jit
def _lookup(xu, xv, uid_t, iid_t):
    mesh = plsc.VectorSubcoreMesh(core_axis_name="c", subcore_axis_name="s")
    f = functools.partial(
        pl.kernel,
        mesh=mesh,
        out_type=jax.ShapeDtypeStruct((B, 2 * D), jnp.float32),
        scratch_types=[
            pltpu.VMEM((BPW,), jnp.int32),
            pltpu.VMEM((BPW,), jnp.int32),
            pltpu.VMEM((CH, TROWS, D), jnp.float32),
            pltpu.VMEM((CH, TROWS, D), jnp.float32),
            pltpu.VMEM((CH, TROWS, D), jnp.float32),
            pltpu.VMEM((CH, TROWS, D), jnp.float32),
            pltpu.VMEM((CH, 2 * D), jnp.float32),
            pltpu.SemaphoreType.DMA,
            pltpu.SemaphoreType.DMA,
        ],
        compiler_params=pltpu.CompilerParams(needs_layout_passes=False),
    )(_body)
    return f(xu, xv, uid_t, iid_t)


def kernel(x, uid_table, iid_table):
    xi = x.astype(jnp.int32)
    ut = uid_table.reshape(125000, TROWS, D)
    it = iid_table[:1000000].reshape(125000, TROWS, D)
    o2 = _lookup(xi[:, 0], xi[:, 1], ut, it)
    return o2.reshape(B, 2, D)
